# baseline (speedup 1.0000x reference)
import jax
import jax.numpy as jnp
from jax import lax
from jax.experimental import pallas as pl
from jax.experimental.pallas import tpu as pltpu
from jax.experimental.pallas import tpu_sc as plsc

Q = 1024
D = 128
N = 100000
K = 25
B = 256
NP = 100352
NB = NP // B
NBP = 400
KB = 2048
GRID = NP // KB
MPB = KB // B
CAP = 128
NEG = -1e30
BIG = 2**30
NWORK = 32
CH = 8
QC = Q // CH
QPW = QC // NWORK


def _mm_kernel(q_ref, kt_ref, s_ref, m_ref):
    i = pl.program_id(0)
    s = lax.dot_general(q_ref[...], kt_ref[...],
                        dimension_numbers=(((1,), (1,)), ((), ())),
                        preferred_element_type=jnp.float32)
    col = i * KB + lax.broadcasted_iota(jnp.int32, (QC, KB), 1)
    s = jnp.where(col < N, s, NEG)
    lane = lax.broadcasted_iota(jnp.int32, (QC, 128), 1)
    m = jnp.full((QC, 128), NEG, jnp.float32)
    for j in range(MPB):
        blk = s[:, j * B:(j + 1) * B]
        s_ref[pl.ds(j * QC, QC), :] = blk
        mj = jnp.max(blk, axis=1, keepdims=True)
        m = jnp.where(lane == j, jnp.broadcast_to(mj, (QC, 128)), m)
    m_ref[...] = m


def _splat_f(x):
    return jnp.full((16,), x, jnp.float32)


def _splat_i(x):
    return jnp.full((16,), x, jnp.int32)


def _topk_sc_kernel(s2_hbm, m_hbm, outs_hbm, outi_hbm,
                    m_v, ridx_v, rows_v, surv_v, survi_v,
                    o_v, oi_v, sem_m, sem_g, sem_o):
    wid = lax.axis_index("s") * 2 + lax.axis_index("c")
    lanes = lax.iota(jnp.int32, 16)

    def phase1(i):
        q = wid * QPW + i
        b = i % 2

        def extract_block(j, carry):
            lv, li, b0, b1 = carry

            def scan(t, c):
                av, ai = c
                v = m_v[b, pl.ds(t * 16, 16)]
                idx = _splat_i(t * 16) + lanes
                avail = (v < _splat_f(lv)) | ((v == _splat_f(lv)) &
                                              (idx > _splat_i(li)))
                vv = jnp.where(avail, v, _splat_f(-jnp.inf))
                gt = (vv > av) | ((vv == av) & (idx < ai))
                return jnp.where(gt, vv, av), jnp.where(gt, idx, ai)

            av, ai = lax.fori_loop(0, NBP // 16, scan,
                                   (_splat_f(-jnp.inf), _splat_i(BIG)))
            mx = jnp.max(av)
            bid = jnp.min(jnp.where(av == _splat_f(mx), ai, _splat_i(BIG)))
            b0 = jnp.where(lanes == _splat_i(j), _splat_i(bid), b0)
            b1 = jnp.where(lanes == _splat_i(j - 16), _splat_i(bid), b1)
            return mx, bid, b0, b1

        t_star, _, b0, b1 = lax.fori_loop(
            0, K, extract_block,
            (jnp.float32(jnp.inf), jnp.int32(-1),
             jnp.zeros((16,), jnp.int32), jnp.zeros((16,), jnp.int32)))

        qs = _splat_i(q)
        ridx_v[b, pl.ds(0, 16)] = b0 * _splat_i(QC) + qs
        ridx_v[b, pl.ds(16, 16)] = b1 * _splat_i(QC) + qs
        pltpu.async_copy(s2_hbm.at[ridx_v.at[b]], rows_v.at[b], sem_g.at[b])
        return t_star, b0, b1

    def phase34(i, t_star, b0, b1):
        q = wid * QPW + i
        b = i % 2

        def init_sv(t, c):
            surv_v[pl.ds(t * 16, 16)] = _splat_f(-jnp.inf)
            survi_v[pl.ds(t * 16, 16)] = _splat_i(0)
            return c

        lax.fori_loop(0, CAP // 16, init_sv, 0)
        pltpu.make_async_copy(s2_hbm.at[ridx_v.at[b]], rows_v.at[b],
                              sem_g.at[b]).wait()
        tsv = _splat_f(t_star)

        def frow(rr, cnt):
            bid0 = jnp.max(jnp.where(lanes == _splat_i(rr), b0, _splat_i(0)))
            bid1 = jnp.max(jnp.where(lanes == _splat_i(rr - 16), b1,
                                     _splat_i(0)))
            gbase = _splat_i(jnp.where(rr < 16, bid0, bid1) * B)

            def fcol(cc, cnt2):
                v = rows_v[b, rr, pl.ds(cc * 16, 16)]
                msk = v >= tsv
                gid = gbase + _splat_i(cc * 16) + lanes
                off = jnp.minimum(cnt2, CAP - 16)
                plsc.store_compressed(surv_v.at[pl.ds(off, 16)], v, mask=msk)
                plsc.store_compressed(survi_v.at[pl.ds(off, 16)], gid,
                                      mask=msk)
                return cnt2 + jnp.max(plsc.all_reduce_population_count(msk))

            return lax.fori_loop(0, B // 16, fcol, cnt)

        lax.fori_loop(0, K, frow, jnp.int32(0))

        def extract_fin(j, carry):
            lv, li, s0, s1, i0, i1 = carry

            def scan(t, c):
                av, ai = c
                v = surv_v[pl.ds(t * 16, 16)]
                gi = survi_v[pl.ds(t * 16, 16)]
                avail = (v < _splat_f(lv)) | ((v == _splat_f(lv)) &
                                              (gi > _splat_i(li)))
                vv = jnp.where(avail, v, _splat_f(-jnp.inf))
                gt = (vv > av) | ((vv == av) & (gi < ai))
                return jnp.where(gt, vv, av), jnp.where(gt, gi, ai)

            av, ai = lax.fori_loop(0, CAP // 16, scan,
                                   (_splat_f(-jnp.inf), _splat_i(BIG)))
            mx = jnp.max(av)
            sid = jnp.min(jnp.where(av == _splat_f(mx), ai, _splat_i(BIG)))
            s0 = jnp.where(lanes == _splat_i(j), _splat_f(mx), s0)
            s1 = jnp.where(lanes == _splat_i(j - 16), _splat_f(mx), s1)
            i0 = jnp.where(lanes == _splat_i(j), _splat_i(sid), i0)
            i1 = jnp.where(lanes == _splat_i(j - 16), _splat_i(sid), i1)
            return mx, sid, s0, s1, i0, i1

        _, _, s0, s1, i0, i1 = lax.fori_loop(
            0, K, extract_fin,
            (jnp.float32(jnp.inf), jnp.int32(-1),
             jnp.zeros((16,), jnp.float32), jnp.zeros((16,), jnp.float32),
             jnp.zeros((16,), jnp.int32), jnp.zeros((16,), jnp.int32)))

        @pl.when(i >= 2)
        def _():
            qp = q - 2
            pltpu.make_async_copy(o_v.at[b], outs_hbm.at[qp],
                                  sem_o.at[b]).wait()
            pltpu.make_async_copy(oi_v.at[b], outi_hbm.at[qp],
                                  sem_o.at[b]).wait()

        o_v[b, pl.ds(0, 16)] = s0
        o_v[b, pl.ds(16, 16)] = s1
        oi_v[b, pl.ds(0, 16)] = i0
        oi_v[b, pl.ds(16, 16)] = i1
        pltpu.async_copy(o_v.at[b], outs_hbm.at[q], sem_o.at[b])
        pltpu.async_copy(oi_v.at[b], outi_hbm.at[q], sem_o.at[b])

    pltpu.sync_copy(m_hbm.at[wid * QPW], m_v.at[0])

    def step(i, carry):
        tp, b0p, b1p = carry

        @pl.when(i + 1 < QPW)
        def _():
            pltpu.async_copy(m_hbm.at[wid * QPW + i + 1],
                             m_v.at[(i + 1) % 2], sem_m)

        t, b0, b1 = phase1(i)

        @pl.when(i > 0)
        def _():
            phase34(i - 1, tp, b0p, b1p)

        @pl.when(i + 1 < QPW)
        def _():
            pltpu.make_async_copy(m_hbm.at[wid * QPW + i + 1],
                                  m_v.at[(i + 1) % 2], sem_m).wait()

        return t, b0, b1

    tl, b0l, b1l = lax.fori_loop(
        0, QPW, step,
        (jnp.float32(0.0), jnp.zeros((16,), jnp.int32),
         jnp.zeros((16,), jnp.int32)))
    phase34(QPW - 1, tl, b0l, b1l)
    for i in (QPW - 2, QPW - 1):
        q = wid * QPW + i
        pltpu.make_async_copy(o_v.at[i % 2], outs_hbm.at[q],
                              sem_o.at[i % 2]).wait()
        pltpu.make_async_copy(oi_v.at[i % 2], outi_hbm.at[q],
                              sem_o.at[i % 2]).wait()


def _mm_chunk(qc, kt):
    return pl.pallas_call(
        _mm_kernel,
        grid=(GRID,),
        in_specs=[pl.BlockSpec((QC, D), lambda i: (0, 0)),
                  pl.BlockSpec((KB, D), lambda i: (i, 0))],
        out_specs=[pl.BlockSpec((QC * MPB, B), lambda i: (i, 0)),
                   pl.BlockSpec((QC, 128), lambda i: (0, i))],
        out_shape=[jax.ShapeDtypeStruct((QC * NB, B), jnp.float32),
                   jax.ShapeDtypeStruct((QC, GRID * 128), jnp.float32)],
    )(qc, kt)


def _topk_chunk(s2, m):
    mb = m.reshape(QC, GRID, 128)[:, :, :MPB].reshape(QC, NB)
    mp = jnp.pad(mb, ((0, 0), (0, NBP - NB)), constant_values=-1e30)
    topk = pl.kernel(
        _topk_sc_kernel,
        out_type=[jax.ShapeDtypeStruct((QC, 32), jnp.float32),
                  jax.ShapeDtypeStruct((QC, 32), jnp.int32)],
        mesh=plsc.VectorSubcoreMesh(core_axis_name="c", subcore_axis_name="s",
                                    num_cores=2, num_subcores=16),
        compiler_params=pltpu.CompilerParams(needs_layout_passes=False),
        scratch_types=[
            pltpu.VMEM((2, NBP), jnp.float32),
            pltpu.VMEM((2, 32), jnp.int32),
            pltpu.VMEM((2, 32, B), jnp.float32),
            pltpu.VMEM((CAP,), jnp.float32),
            pltpu.VMEM((CAP,), jnp.int32),
            pltpu.VMEM((2, 32), jnp.float32),
            pltpu.VMEM((2, 32), jnp.int32),
            pltpu.SemaphoreType.DMA,
            pltpu.SemaphoreType.DMA((2,)),
            pltpu.SemaphoreType.DMA((2,)),
        ],
    )
    return topk(s2, mp)


def kernel(queries, keys):
    parts = []
    for c in range(CH):
        s, m = _mm_chunk(lax.slice(queries, (c * QC, 0), ((c + 1) * QC, D)),
                         keys)
        parts.append(_topk_chunk(s, m))
    outs = jnp.concatenate([p[0] for p in parts], axis=0)
    outi = jnp.concatenate([p[1] for p in parts], axis=0)
    return outs[:, :K], outi[:, :K]


# speedup vs baseline: 1.3277x; 1.3277x over previous
import jax
import jax.numpy as jnp
from jax import lax
from jax.experimental import pallas as pl
from jax.experimental.pallas import tpu as pltpu
from jax.experimental.pallas import tpu_sc as plsc

Q = 1024
D = 128
N = 100000
K = 25
B = 256
NP = 100352
NB = NP // B
NBP = 400
KB = 2048
GRID = NP // KB
MPB = KB // B
CAP = 128
NEG = -1e30
BIG = 2**30
NWORK = 32
CH = 4
QC = Q // CH
QPW = QC // NWORK


def _mm_kernel(q_ref, kt_ref, s_ref, m_ref):
    i = pl.program_id(0)
    s = lax.dot_general(q_ref[...], kt_ref[...],
                        dimension_numbers=(((1,), (1,)), ((), ())),
                        preferred_element_type=jnp.float32)
    col = i * KB + lax.broadcasted_iota(jnp.int32, (QC, KB), 1)
    s = jnp.where(col < N, s, NEG)
    lane = lax.broadcasted_iota(jnp.int32, (QC, 128), 1)
    m = jnp.full((QC, 128), NEG, jnp.float32)
    for j in range(MPB):
        blk = s[:, j * B:(j + 1) * B]
        s_ref[pl.ds(j * QC, QC), :] = blk
        mj = jnp.max(blk, axis=1, keepdims=True)
        m = jnp.where(lane == j, jnp.broadcast_to(mj, (QC, 128)), m)
    m_ref[...] = m


def _splat_f(x):
    return jnp.full((16,), x, jnp.float32)


def _splat_i(x):
    return jnp.full((16,), x, jnp.int32)


def _topk_sc_kernel(s2_hbm, m_hbm, outs_hbm, outi_hbm,
                    m_v, ridx_v, rows_v, surv_v, survi_v,
                    o_v, oi_v, sem_m, sem_g, sem_o):
    wid = lax.axis_index("s") * 2 + lax.axis_index("c")
    lanes = lax.iota(jnp.int32, 16)

    def phase1(i):
        q = wid * QPW + i
        b = i % 2

        def extract_block(j, carry):
            lv, li, b0, b1 = carry

            def scan(t, c):
                av, ai = c
                v = m_v[b, pl.ds(t * 16, 16)]
                idx = _splat_i(t * 16) + lanes
                avail = (v < _splat_f(lv)) | ((v == _splat_f(lv)) &
                                              (idx > _splat_i(li)))
                vv = jnp.where(avail, v, _splat_f(-jnp.inf))
                gt = (vv > av) | ((vv == av) & (idx < ai))
                return jnp.where(gt, vv, av), jnp.where(gt, idx, ai)

            av, ai = lax.fori_loop(0, NBP // 16, scan,
                                   (_splat_f(-jnp.inf), _splat_i(BIG)))
            mx = jnp.max(av)
            bid = jnp.min(jnp.where(av == _splat_f(mx), ai, _splat_i(BIG)))
            b0 = jnp.where(lanes == _splat_i(j), _splat_i(bid), b0)
            b1 = jnp.where(lanes == _splat_i(j - 16), _splat_i(bid), b1)
            return mx, bid, b0, b1

        t_star, _, b0, b1 = lax.fori_loop(
            0, K, extract_block,
            (jnp.float32(jnp.inf), jnp.int32(-1),
             jnp.zeros((16,), jnp.int32), jnp.zeros((16,), jnp.int32)))

        qs = _splat_i(q)
        ridx_v[b, pl.ds(0, 16)] = b0 * _splat_i(QC) + qs
        ridx_v[b, pl.ds(16, 16)] = b1 * _splat_i(QC) + qs
        pltpu.async_copy(s2_hbm.at[ridx_v.at[b]], rows_v.at[b], sem_g.at[b])
        return t_star, b0, b1

    def phase34(i, t_star, b0, b1):
        q = wid * QPW + i
        b = i % 2

        def init_sv(t, c):
            surv_v[pl.ds(t * 16, 16)] = _splat_f(-jnp.inf)
            survi_v[pl.ds(t * 16, 16)] = _splat_i(0)
            return c

        lax.fori_loop(0, CAP // 16, init_sv, 0)
        pltpu.make_async_copy(s2_hbm.at[ridx_v.at[b]], rows_v.at[b],
                              sem_g.at[b]).wait()
        tsv = _splat_f(t_star)

        def frow(rr, cnt):
            bid0 = jnp.max(jnp.where(lanes == _splat_i(rr), b0, _splat_i(0)))
            bid1 = jnp.max(jnp.where(lanes == _splat_i(rr - 16), b1,
                                     _splat_i(0)))
            gbase = _splat_i(jnp.where(rr < 16, bid0, bid1) * B)

            def fcol(cc, cnt2):
                v = rows_v[b, rr, pl.ds(cc * 16, 16)]
                msk = v >= tsv
                gid = gbase + _splat_i(cc * 16) + lanes
                off = jnp.minimum(cnt2, CAP - 16)
                plsc.store_compressed(surv_v.at[pl.ds(off, 16)], v, mask=msk)
                plsc.store_compressed(survi_v.at[pl.ds(off, 16)], gid,
                                      mask=msk)
                return cnt2 + jnp.max(plsc.all_reduce_population_count(msk))

            return lax.fori_loop(0, B // 16, fcol, cnt)

        lax.fori_loop(0, K, frow, jnp.int32(0))

        def extract_fin(j, carry):
            lv, li, s0, s1, i0, i1 = carry

            def scan(t, c):
                av, ai = c
                v = surv_v[pl.ds(t * 16, 16)]
                gi = survi_v[pl.ds(t * 16, 16)]
                avail = (v < _splat_f(lv)) | ((v == _splat_f(lv)) &
                                              (gi > _splat_i(li)))
                vv = jnp.where(avail, v, _splat_f(-jnp.inf))
                gt = (vv > av) | ((vv == av) & (gi < ai))
                return jnp.where(gt, vv, av), jnp.where(gt, gi, ai)

            av, ai = lax.fori_loop(0, CAP // 16, scan,
                                   (_splat_f(-jnp.inf), _splat_i(BIG)))
            mx = jnp.max(av)
            sid = jnp.min(jnp.where(av == _splat_f(mx), ai, _splat_i(BIG)))
            s0 = jnp.where(lanes == _splat_i(j), _splat_f(mx), s0)
            s1 = jnp.where(lanes == _splat_i(j - 16), _splat_f(mx), s1)
            i0 = jnp.where(lanes == _splat_i(j), _splat_i(sid), i0)
            i1 = jnp.where(lanes == _splat_i(j - 16), _splat_i(sid), i1)
            return mx, sid, s0, s1, i0, i1

        _, _, s0, s1, i0, i1 = lax.fori_loop(
            0, K, extract_fin,
            (jnp.float32(jnp.inf), jnp.int32(-1),
             jnp.zeros((16,), jnp.float32), jnp.zeros((16,), jnp.float32),
             jnp.zeros((16,), jnp.int32), jnp.zeros((16,), jnp.int32)))

        @pl.when(i >= 2)
        def _():
            qp = q - 2
            pltpu.make_async_copy(o_v.at[b], outs_hbm.at[qp],
                                  sem_o.at[b]).wait()
            pltpu.make_async_copy(oi_v.at[b], outi_hbm.at[qp],
                                  sem_o.at[b]).wait()

        o_v[b, pl.ds(0, 16)] = s0
        o_v[b, pl.ds(16, 16)] = s1
        oi_v[b, pl.ds(0, 16)] = i0
        oi_v[b, pl.ds(16, 16)] = i1
        pltpu.async_copy(o_v.at[b], outs_hbm.at[q], sem_o.at[b])
        pltpu.async_copy(oi_v.at[b], outi_hbm.at[q], sem_o.at[b])

    pltpu.sync_copy(m_hbm.at[wid * QPW], m_v.at[0])

    def step(i, carry):
        tp, b0p, b1p = carry

        @pl.when(i + 1 < QPW)
        def _():
            pltpu.async_copy(m_hbm.at[wid * QPW + i + 1],
                             m_v.at[(i + 1) % 2], sem_m)

        t, b0, b1 = phase1(i)

        @pl.when(i > 0)
        def _():
            phase34(i - 1, tp, b0p, b1p)

        @pl.when(i + 1 < QPW)
        def _():
            pltpu.make_async_copy(m_hbm.at[wid * QPW + i + 1],
                                  m_v.at[(i + 1) % 2], sem_m).wait()

        return t, b0, b1

    tl, b0l, b1l = lax.fori_loop(
        0, QPW, step,
        (jnp.float32(0.0), jnp.zeros((16,), jnp.int32),
         jnp.zeros((16,), jnp.int32)))
    phase34(QPW - 1, tl, b0l, b1l)
    for i in (QPW - 2, QPW - 1):
        q = wid * QPW + i
        pltpu.make_async_copy(o_v.at[i % 2], outs_hbm.at[q],
                              sem_o.at[i % 2]).wait()
        pltpu.make_async_copy(oi_v.at[i % 2], outi_hbm.at[q],
                              sem_o.at[i % 2]).wait()


def _mm_chunk(qc, kt):
    return pl.pallas_call(
        _mm_kernel,
        grid=(GRID,),
        in_specs=[pl.BlockSpec((QC, D), lambda i: (0, 0)),
                  pl.BlockSpec((KB, D), lambda i: (i, 0))],
        out_specs=[pl.BlockSpec((QC * MPB, B), lambda i: (i, 0)),
                   pl.BlockSpec((QC, 128), lambda i: (0, i))],
        out_shape=[jax.ShapeDtypeStruct((QC * NB, B), jnp.float32),
                   jax.ShapeDtypeStruct((QC, GRID * 128), jnp.float32)],
    )(qc, kt)


def _topk_chunk(s2, m):
    mb = m.reshape(QC, GRID, 128)[:, :, :MPB].reshape(QC, NB)
    mp = jnp.pad(mb, ((0, 0), (0, NBP - NB)), constant_values=-1e30)
    topk = pl.kernel(
        _topk_sc_kernel,
        out_type=[jax.ShapeDtypeStruct((QC, 32), jnp.float32),
                  jax.ShapeDtypeStruct((QC, 32), jnp.int32)],
        mesh=plsc.VectorSubcoreMesh(core_axis_name="c", subcore_axis_name="s",
                                    num_cores=2, num_subcores=16),
        compiler_params=pltpu.CompilerParams(needs_layout_passes=False),
        scratch_types=[
            pltpu.VMEM((2, NBP), jnp.float32),
            pltpu.VMEM((2, 32), jnp.int32),
            pltpu.VMEM((2, 32, B), jnp.float32),
            pltpu.VMEM((CAP,), jnp.float32),
            pltpu.VMEM((CAP,), jnp.int32),
            pltpu.VMEM((2, 32), jnp.float32),
            pltpu.VMEM((2, 32), jnp.int32),
            pltpu.SemaphoreType.DMA,
            pltpu.SemaphoreType.DMA((2,)),
            pltpu.SemaphoreType.DMA((2,)),
        ],
    )
    return topk(s2, mp)


def kernel(queries, keys):
    parts = []
    for c in range(CH):
        s, m = _mm_chunk(lax.slice(queries, (c * QC, 0), ((c + 1) * QC, D)),
                         keys)
        parts.append(_topk_chunk(s, m))
    outs = jnp.concatenate([p[0] for p in parts], axis=0)
    outi = jnp.concatenate([p[1] for p in parts], axis=0)
    return outs[:, :K], outi[:, :K]
